# Initial kernel scaffold; baseline (speedup 1.0000x reference)
import jax
import jax.numpy as jnp
from jax.experimental import pallas as pl


def kernel(u_id, i_id, features, user_emb, user_bias, item_emb, item_bias, feat_u, feat_i, mean):
    raise NotImplementedError("write your pallas kernel here")


# speedup vs baseline: 1.6565x; 1.6565x over previous
import functools

import jax
import jax.numpy as jnp
from jax import lax
from jax.experimental import pallas as pl
from jax.experimental.pallas import tpu as pltpu
from jax.experimental.pallas import tpu_sc as plsc

B = 16384
EMB = 64
NF = 26
FEMB = 32
FEAT_VOCAB = 1000
NUM_ITEMS = 100000

NW = 32
PER_W = B // NW
C = 32
CHUNKS = PER_W // C
CF = C * NF
IDX_W = 64
NDMA = CF // IDX_W

_mesh = plsc.VectorSubcoreMesh(core_axis_name="c", subcore_axis_name="s")


@functools.partial(
    pl.kernel,
    out_type=jax.ShapeDtypeStruct((B,), jnp.float32),
    mesh=_mesh,
    compiler_params=pltpu.CompilerParams(use_tc_tiling_on_sc=False),
    scratch_types=[
        pltpu.VMEM((C,), jnp.int32),
        pltpu.VMEM((C,), jnp.int32),
        pltpu.VMEM((CF,), jnp.int32),
        pltpu.VMEM((NDMA, IDX_W), jnp.int32),
        pltpu.VMEM((NDMA, IDX_W), jnp.int32),
        pltpu.VMEM((C, EMB), jnp.float32),
        pltpu.VMEM((C, EMB), jnp.float32),
        pltpu.VMEM((C,), jnp.float32),
        pltpu.VMEM((C,), jnp.float32),
        pltpu.VMEM((CF, FEMB), jnp.float32),
        pltpu.VMEM((CF, FEMB), jnp.float32),
        pltpu.VMEM((C,), jnp.float32),
        pltpu.VMEM((256,), jnp.float32),
        pltpu.VMEM((16,), jnp.float32),
        pltpu.SemaphoreType.DMA,
    ],
)
def _mf_sc(u_id, i_id, feats, user_emb, user_bias, item_emb, item_bias,
           fu_tab, fi_tab, mean, out_hbm,
           u_v, i_v, f_v, fu_idx, fi_idx,
           U_v, I_v, bu_v, bi_v, FU_v, FI_v, out_v, P_v, mean_v, sem):
    wid = lax.axis_index("s") * 2 + lax.axis_index("c")
    base0 = wid * PER_W

    pltpu.sync_copy(mean, mean_v)

    def chunk(g, carry):
        base = base0 + g * C
        pltpu.sync_copy(u_id.at[pl.ds(base, C)], u_v)
        pltpu.sync_copy(i_id.at[pl.ds(base, C)], i_v)
        pltpu.sync_copy(feats.at[pl.ds(base * NF, CF)], f_v)

        for j in range(CF // 16):
            f = j * 16 // C
            s = pl.ds(j * 16, 16)
            r = j // (IDX_W // 16)
            cs = pl.ds((j % (IDX_W // 16)) * 16, 16)
            fu_idx[r, cs] = f_v[s] + (f * FEAT_VOCAB)
            fi_idx[r, cs] = i_v[pl.ds((j % (C // 16)) * 16, 16)] + (f * NUM_ITEMS)

        cps = [
            pltpu.async_copy(user_emb.at[u_v], U_v, sem),
            pltpu.async_copy(item_emb.at[i_v], I_v, sem),
            pltpu.async_copy(user_bias.at[u_v], bu_v, sem),
            pltpu.async_copy(item_bias.at[i_v], bi_v, sem),
        ]
        for j in range(NDMA):
            d = pl.ds(j * IDX_W, IDX_W)
            cps.append(pltpu.async_copy(fu_tab.at[fu_idx.at[j]], FU_v.at[d], sem))
            cps.append(pltpu.async_copy(fi_tab.at[fi_idx.at[j]], FI_v.at[d], sem))
        for cp in cps:
            cp.wait()

        lanes = lax.iota(jnp.int32, 16)

        def elt(b, _):
            b2 = b & 15
            acc = U_v[b, pl.ds(0, 16)] * I_v[b, pl.ds(0, 16)]
            for k in range(1, EMB // 16):
                cs = pl.ds(k * 16, 16)
                acc = acc + U_v[b, cs] * I_v[b, cs]
            for f in range(NF):
                row = f * C + b
                for h in range(FEMB // 16):
                    cs = pl.ds(h * 16, 16)
                    acc = acc + FU_v[row, cs] * FI_v[row, cs]
            br = ((b2 & 1) << 3) | ((b2 & 2) << 1) | ((b2 & 4) >> 1) | ((b2 & 8) >> 3)
            P_v[pl.ds(br * 16, 16)] = acc
            return _

        for g2 in range(C // 16):
            lax.fori_loop(g2 * 16, (g2 + 1) * 16, elt, 0, unroll=False)
            for rnd, (w, n) in enumerate([(8, 8), (4, 4), (2, 2), (1, 1)]):
                for k in range(n):
                    a0 = 32 * k
                    t1 = P_v[pl.ds(a0, 16)] + P_v[pl.ds(a0 + w, 16)]
                    t2 = P_v[pl.ds(a0 + 16 - w, 16)] + P_v[pl.ds(a0 + 16, 16)]
                    sel = (lanes & (2 * w - 1)) < w
                    q = jnp.where(sel, t1, t2)
                    if rnd < 3:
                        P_v[pl.ds(16 * k, 16)] = q
            s16 = pl.ds(g2 * 16, 16)
            out_v[s16] = q + bu_v[s16] + bi_v[s16] + mean_v[pl.ds(0, 16)]

        pltpu.sync_copy(out_v, out_hbm.at[pl.ds(base, C)])
        return carry

    lax.fori_loop(0, CHUNKS, chunk, 0, unroll=False)


def kernel(u_id, i_id, features, user_emb, user_bias, item_emb, item_bias,
           feat_u, feat_i, mean):
    u_id = u_id.astype(jnp.int32)
    i_id = i_id.astype(jnp.int32)
    feats = (features.astype(jnp.int32)
             .reshape(B // C, C, NF).transpose(0, 2, 1).reshape(-1))
    fu_tab = feat_u.reshape(NF * FEAT_VOCAB, FEMB)
    fi_tab = feat_i.reshape(NF * NUM_ITEMS, FEMB)
    ub = user_bias.reshape(-1)
    ib = item_bias.reshape(-1)
    mean16 = jnp.broadcast_to(mean, (16,))
    return _mf_sc(u_id, i_id, feats, user_emb, ub, item_emb,
                  ib, fu_tab, fi_tab, mean16)
